# baseline (speedup 1.0000x reference)
import functools

import jax
import jax.numpy as jnp
from jax import lax
from jax.experimental import pallas as pl
from jax.experimental.pallas import tpu as pltpu
from jax.experimental.pallas import tpu_sc as plsc

NC, NS, L = 2, 16, 16
NW = NC * NS
S_DIM = 1000000
A_DIM = 32
B = 16384
LR = 0.01

MAIN_BLKS = 7812
MAIN = MAIN_BLKS * 128
TAIL = S_DIM - MAIN
SA_BLK = 20
WA = SA_BLK * 128
TRIPS_A = 13
SB_BLK = 16
WB = SB_BLK * 128
TRIPS_B = 16
NBLK = B // L
CAP = B

_mesh = plsc.VectorSubcoreMesh(
    core_axis_name="c", subcore_axis_name="s", num_cores=NC, num_subcores=NS)
_params = pltpu.CompilerParams(needs_layout_passes=False)


def _iota16():
    return lax.iota(jnp.int32, L)


def _wid():
    return lax.axis_index("s") * NC + lax.axis_index("c")


def _scalar(v16):
    return lax.reduce_max(v16, (0,))


def _region(w):
    lo_b = (w * MAIN_BLKS) // NW
    hi_b = ((w + 1) * MAIN_BLKS) // NW
    return lo_b, hi_b


def _chunk_bounds(c, lo_b, hi_b, s_blk, trips):
    n = hi_b - lo_b
    st_b = lo_b + jnp.minimum(c * s_blk, n - s_blk)
    nx_b = jnp.where(c + 1 >= trips, hi_b,
                     lo_b + jnp.minimum((c + 1) * s_blk, n - s_blk))
    return st_b * 128, st_b * 128, nx_b * 128


def _dedup_tot(fidx, val, valid, ksr, csr, vsr):
    big = jnp.where(valid, fidx, jnp.int32(0x7FFFFFFF))
    v = jnp.where(valid, val, 0.0)
    ks, vs = plsc.sort_key_val(big, v)
    io = _iota16()
    ksr[pl.ds(0, L)] = ks
    prev = plsc.load_gather(ksr, [jnp.maximum(io - 1, 0)])
    nxt = plsc.load_gather(ksr, [jnp.minimum(io + 1, L - 1)])
    rs = (io == 0) | (ks != prev)
    re = (io == L - 1) | (ks != nxt)
    cs = plsc.cumsum(vs)
    csr[pl.ds(0, L)] = cs
    vsr[pl.ds(0, L)] = vs
    bs = plsc.cummax(jnp.where(rs, io, 0))
    base = plsc.load_gather(csr, [bs]) - plsc.load_gather(vsr, [bs])
    tot = cs - base
    ok = re & (ks != jnp.int32(0x7FFFFFFF))
    return ks, tot, ok


@functools.partial(
    pl.kernel,
    out_type=jax.ShapeDtypeStruct((NC * B,), jnp.float32),
    mesh=_mesh,
    scratch_types=[
        pltpu.VMEM_SHARED((B,), jnp.float32),
        pltpu.VMEM((A_DIM, WA), jnp.float32),
        pltpu.VMEM((B,), jnp.int32),
        pltpu.VMEM((CAP,), jnp.int32),
        pltpu.VMEM((4, 128), jnp.int32),
        pltpu.VMEM((4, 128), jnp.float32),
        pltpu.VMEM((TAIL * A_DIM,), jnp.float32),
        pltpu.VMEM((1024,), jnp.float32),
        pltpu.VMEM((L,), jnp.int32),
        pltpu.VMEM((L,), jnp.float32),
        pltpu.SemaphoreType.DMA,
    ],
    compiler_params=_params,
)
def _passA(ns_h, qtail_h, qT, qn_out,
           qn_sh, buf, nsv, own_ns, sidx, sval, tailv, zbuf, tidx, tval, sem):
    w = _wid()
    cid = lax.axis_index("c")
    sid = lax.axis_index("s")
    pltpu.sync_copy(ns_h, nsv)

    lo_b, hi_b = _region(w)
    lo_s, hi_s = lo_b * 128, hi_b * 128

    def build(b, cur):
        o = b * L
        ns = nsv[pl.ds(o, L)]
        mn = (ns >= lo_s) & (ns < hi_s)
        plsc.store_compressed(own_ns.at[pl.ds(cur, L)], o + _iota16(),
                              mask=mn)
        return cur + _scalar(plsc.all_reduce_population_count(mn))
    cnt_ns = lax.fori_loop(0, NBLK, build, 0)

    def z(b, _):
        zbuf[pl.ds(b * L, L)] = jnp.zeros((L,), jnp.float32)
        return 0
    lax.fori_loop(0, 1024 // L, z, 0)
    pltpu.sync_copy(zbuf, qn_sh.at[pl.ds(sid * 1024, 1024)])
    plsc.subcore_barrier()

    zero16 = jnp.zeros((L,), jnp.float32)
    zero16i = jnp.zeros((L,), jnp.int32)
    tn32 = ((cnt_ns + (32 * L - 1)) // (32 * L)) * 32

    def chunk(c, _):
        s0, can_lo, can_hi = _chunk_bounds(c, lo_b, hi_b, SA_BLK, TRIPS_A)
        pltpu.sync_copy(qT.at[pl.ds(0, A_DIM), pl.ds(s0, WA)], buf)

        def nblk_body(b, _):
            g = (b % 32) // 8
            r = (b % 8) * L
            ids = jnp.where((b * L + _iota16()) < cnt_ns,
                            own_ns[pl.ds(b * L, L)], 0)
            ns = plsc.load_gather(nsv, [ids])
            msk = ((b * L + _iota16()) < cnt_ns) \
                & (ns >= can_lo) & (ns < can_hi)

            def active(_):
                col = jnp.where(msk, ns - s0, 0)
                m = plsc.load_gather(buf, [zero16i, col])
                for a in range(1, A_DIM):
                    m = jnp.maximum(m, plsc.load_gather(
                        buf, [jnp.full((L,), a, jnp.int32), col]))
                sidx[g, pl.ds(r, L)] = jnp.where(msk, ids, 0)
                sval[g, pl.ds(r, L)] = jnp.where(msk, m, 0.0)
                return 0

            def idle(_):
                sidx[g, pl.ds(r, L)] = zero16i
                sval[g, pl.ds(r, L)] = zero16
                return 0

            lax.cond(jnp.any(msk), active, idle, 0)

            @pl.when(b % 32 == 31)
            def _():
                cps = [pltpu.async_copy(sval.at[j], qn_sh.at[sidx.at[j]],
                                        sem, add=True) for j in range(4)]
                for cp in cps:
                    cp.wait()
            return 0

        lax.fori_loop(0, tn32, nblk_body, 0)
        return 0

    lax.fori_loop(0, TRIPS_A, chunk, 0)

    @pl.when(w == 0)
    def _tail():
        pltpu.sync_copy(qtail_h, tailv)

        def blk(b, _):
            o = b * L
            ids = o + _iota16()
            ns = nsv[pl.ds(o, L)]
            mn = ns >= MAIN

            def do_ns(_):
                col = jnp.where(mn, (ns - MAIN) * A_DIM, 0)
                m = plsc.load_gather(tailv, [col])
                for a in range(1, A_DIM):
                    m = jnp.maximum(m, plsc.load_gather(tailv, [col + a]))
                tidx[pl.ds(0, L)] = jnp.where(mn, ids, 0)
                tval[pl.ds(0, L)] = jnp.where(mn, m, 0.0)
                pltpu.sync_copy(tval, qn_sh.at[tidx], add=True)
                return 0
            lax.cond(jnp.any(mn), do_ns, lambda _: 0, 0)
            return 0

        lax.fori_loop(0, NBLK, blk, 0)

    plsc.subcore_barrier()
    pltpu.sync_copy(qn_sh.at[pl.ds(sid * 1024, 1024)],
                    qn_out.at[pl.ds(cid * B + sid * 1024, 1024)])


PB = B // NW


@functools.partial(
    pl.kernel,
    out_type=(jax.ShapeDtypeStruct((B,), jnp.int32),
              jax.ShapeDtypeStruct((B,), jnp.float32)),
    mesh=_mesh,
    scratch_types=[
        pltpu.VMEM((PB,), jnp.int32),
        pltpu.VMEM((PB,), jnp.int32),
        pltpu.VMEM((PB,), jnp.float32),
        pltpu.VMEM((PB,), jnp.float32),
        pltpu.VMEM((PB,), jnp.float32),
        pltpu.VMEM((PB,), jnp.int32),
        pltpu.VMEM((PB,), jnp.float32),
    ],
    compiler_params=_params,
)
def _passB1(st_h, ac_h, rw_h, qn2_h, key_out, tdt_out,
            stv, acv, rwv, qna, qnb, kv, tv):
    w = _wid()
    base = w * PB
    pltpu.sync_copy(st_h.at[pl.ds(base, PB)], stv)
    pltpu.sync_copy(ac_h.at[pl.ds(base, PB)], acv)
    pltpu.sync_copy(rw_h.at[pl.ds(base, PB)], rwv)
    pltpu.sync_copy(qn2_h.at[pl.ds(base, PB)], qna)
    pltpu.sync_copy(qn2_h.at[pl.ds(B + base, PB)], qnb)

    def blk(b, _):
        o = b * L
        qn = qna[pl.ds(o, L)] + qnb[pl.ds(o, L)]
        tv[pl.ds(o, L)] = rwv[pl.ds(o, L)] + qn
        kv[pl.ds(o, L)] = stv[pl.ds(o, L)] * A_DIM + acv[pl.ds(o, L)]
        return 0
    lax.fori_loop(0, PB // L, blk, 0)

    pltpu.sync_copy(kv, key_out.at[pl.ds(base, PB)])
    pltpu.sync_copy(tv, tdt_out.at[pl.ds(base, PB)])


@functools.partial(
    pl.kernel,
    out_type=(jax.ShapeDtypeStruct((A_DIM, S_DIM), jnp.float32),
              jax.ShapeDtypeStruct((TAIL * A_DIM,), jnp.float32)),
    mesh=_mesh,
    scratch_types=[
        pltpu.VMEM((A_DIM, WB), jnp.float32),
        pltpu.VMEM((B,), jnp.int32),
        pltpu.VMEM((B,), jnp.float32),
        pltpu.VMEM((CAP,), jnp.int32),
        pltpu.VMEM((TAIL * A_DIM,), jnp.float32),
        pltpu.VMEM((L,), jnp.int32),
        pltpu.VMEM((L,), jnp.float32),
        pltpu.VMEM((L,), jnp.float32),
    ],
    compiler_params=_params,
)
def _passB2(qT, qtail_h, key_h, tdt_h, outT, tail_out, buf, keyv, tdtv,
            own_st, tailv, ksr, csr, vsr):
    w = _wid()
    pltpu.sync_copy(key_h, keyv)
    pltpu.sync_copy(tdt_h, tdtv)

    lo_b, hi_b = _region(w)
    klo0, khi0 = lo_b * 128 * A_DIM, hi_b * 128 * A_DIM

    def build(b, cur):
        o = b * L
        key = keyv[pl.ds(o, L)]
        msk = (key >= klo0) & (key < khi0)
        plsc.store_compressed(own_st.at[pl.ds(cur, L)], o + _iota16(),
                              mask=msk)
        return cur + _scalar(plsc.all_reduce_population_count(msk))
    cnt = lax.fori_loop(0, NBLK, build, 0)
    nblk = (cnt + L - 1) // L

    def chunk(c, _):
        s0, can_lo, can_hi = _chunk_bounds(c, lo_b, hi_b, SB_BLK, TRIPS_B)
        pltpu.sync_copy(qT.at[pl.ds(0, A_DIM), pl.ds(s0, WB)], buf)
        klo, khi = can_lo * A_DIM, can_hi * A_DIM

        def read_blk(b, _):
            pos = b * L + _iota16()
            live = pos < cnt
            ids = jnp.where(live, own_st[pl.ds(b * L, L)], 0)
            key = plsc.load_gather(keyv, [ids])
            msk = live & (key >= klo) & (key < khi)

            def active(_):
                st = lax.shift_right_logical(key, 5)
                ac = key & (A_DIM - 1)
                col = jnp.where(msk, st - s0, 0)
                qsa = plsc.load_gather(buf, [ac, col])
                tdt = plsc.load_gather(tdtv, [ids])
                plsc.store_scatter(tdtv, [ids], LR * (tdt - qsa), mask=msk)
                return 0
            lax.cond(jnp.any(msk), active, lambda _: 0, 0)
            return 0

        lax.fori_loop(0, nblk, read_blk, 0)

        def wr_blk(b, _):
            pos = b * L + _iota16()
            live = pos < cnt
            ids = jnp.where(live, own_st[pl.ds(b * L, L)], 0)
            key = plsc.load_gather(keyv, [ids])
            msk = live & (key >= klo) & (key < khi)

            def active(_):
                d = plsc.load_gather(tdtv, [ids])
                st = lax.shift_right_logical(key, 5)
                ac = key & (A_DIM - 1)
                fidx = ac * WB + (st - s0)
                ks, tot, ok = _dedup_tot(fidx, d, msk, ksr, csr, vsr)
                q, r = lax.shift_right_logical(ks, 11), ks & (WB - 1)
                plsc.addupdate_scatter(
                    buf, [jnp.where(ok, q, 0), jnp.where(ok, r, 0)],
                    tot, mask=ok)
                return 0
            lax.cond(jnp.any(msk), active, lambda _: 0, 0)
            return 0

        lax.fori_loop(0, nblk, wr_blk, 0)
        pltpu.sync_copy(buf, outT.at[pl.ds(0, A_DIM), pl.ds(s0, WB)])
        return 0

    lax.fori_loop(0, TRIPS_B, chunk, 0)

    @pl.when(w == 0)
    def _tail():
        pltpu.sync_copy(qtail_h, tailv)

        def read_blk(b, _):
            o = b * L
            key = keyv[pl.ds(o, L)]
            msk = key >= MAIN * A_DIM

            def active(_):
                idx = jnp.where(msk, key - MAIN * A_DIM, 0)
                qsa = plsc.load_gather(tailv, [idx])
                tdt = tdtv[pl.ds(o, L)]
                tdtv[pl.ds(o, L)] = jnp.where(msk, LR * (tdt - qsa), tdt)
                return 0
            lax.cond(jnp.any(msk), active, lambda _: 0, 0)
            return 0

        lax.fori_loop(0, NBLK, read_blk, 0)

        def wr_blk(b, _):
            key = keyv[pl.ds(b * L, L)]
            msk = key >= MAIN * A_DIM

            def active(_):
                d = tdtv[pl.ds(b * L, L)]
                fidx = key - MAIN * A_DIM
                ks, tot, ok = _dedup_tot(fidx, d, msk, ksr, csr, vsr)
                plsc.addupdate_scatter(tailv, [jnp.where(ok, ks, 0)], tot,
                                       mask=ok)
                return 0
            lax.cond(jnp.any(msk), active, lambda _: 0, 0)
            return 0

        lax.fori_loop(0, NBLK, wr_blk, 0)
        pltpu.sync_copy(tailv, tail_out)


def kernel(state, action, reward, next_state, instruction, q_table):
    del instruction
    st = state.astype(jnp.int32)
    ac = action.astype(jnp.int32)
    ns = next_state.astype(jnp.int32)
    qT = q_table.T
    qtail = q_table[MAIN:, :].reshape(-1)
    qn2 = _passA(ns, qtail, qT)
    keys, tdt = _passB1(st, ac, reward, qn2)
    outT, tail_out = _passB2(qT, qtail, keys, tdt)
    out = lax.dynamic_update_slice(outT.T, tail_out.reshape(TAIL, A_DIM),
                                   (MAIN, 0))
    return out


# speedup vs baseline: 1.0318x; 1.0318x over previous
import functools

import jax
import jax.numpy as jnp
from jax import lax
from jax.experimental import pallas as pl
from jax.experimental.pallas import tpu as pltpu
from jax.experimental.pallas import tpu_sc as plsc

NC, NS, L = 2, 16, 16
NW = NC * NS
S_DIM = 1000000
A_DIM = 32
B = 16384
LR = 0.01

MAIN_BLKS = 7812
MAIN = MAIN_BLKS * 128
TAIL = S_DIM - MAIN
SA_BLK = 22
WA = SA_BLK * 128
TRIPS_A = 12
SB_BLK = 18
WB = SB_BLK * 128
TRIPS_B = 14
NBLK = B // L
CAP = B

_mesh = plsc.VectorSubcoreMesh(
    core_axis_name="c", subcore_axis_name="s", num_cores=NC, num_subcores=NS)
_params = pltpu.CompilerParams(needs_layout_passes=False)


def _iota16():
    return lax.iota(jnp.int32, L)


def _wid():
    return lax.axis_index("s") * NC + lax.axis_index("c")


def _scalar(v16):
    return lax.reduce_max(v16, (0,))


def _region(w):
    lo_b = (w * MAIN_BLKS) // NW
    hi_b = ((w + 1) * MAIN_BLKS) // NW
    return lo_b, hi_b


def _chunk_bounds(c, lo_b, hi_b, s_blk, trips):
    n = hi_b - lo_b
    st_b = lo_b + jnp.minimum(c * s_blk, n - s_blk)
    nx_b = jnp.where(c + 1 >= trips, hi_b,
                     lo_b + jnp.minimum((c + 1) * s_blk, n - s_blk))
    return st_b * 128, st_b * 128, nx_b * 128


def _dedup_tot(fidx, val, valid, ksr, csr, vsr):
    big = jnp.where(valid, fidx, jnp.int32(0x7FFFFFFF))
    v = jnp.where(valid, val, 0.0)
    ks, vs = plsc.sort_key_val(big, v)
    io = _iota16()
    ksr[pl.ds(0, L)] = ks
    prev = plsc.load_gather(ksr, [jnp.maximum(io - 1, 0)])
    nxt = plsc.load_gather(ksr, [jnp.minimum(io + 1, L - 1)])
    rs = (io == 0) | (ks != prev)
    re = (io == L - 1) | (ks != nxt)
    cs = plsc.cumsum(vs)
    csr[pl.ds(0, L)] = cs
    vsr[pl.ds(0, L)] = vs
    bs = plsc.cummax(jnp.where(rs, io, 0))
    base = plsc.load_gather(csr, [bs]) - plsc.load_gather(vsr, [bs])
    tot = cs - base
    ok = re & (ks != jnp.int32(0x7FFFFFFF))
    return ks, tot, ok


@functools.partial(
    pl.kernel,
    out_type=jax.ShapeDtypeStruct((NC * B,), jnp.float32),
    mesh=_mesh,
    scratch_types=[
        pltpu.VMEM_SHARED((B,), jnp.float32),
        pltpu.VMEM((A_DIM, WA), jnp.float32),
        pltpu.VMEM((B,), jnp.int32),
        pltpu.VMEM((CAP,), jnp.int32),
        pltpu.VMEM((4, 128), jnp.int32),
        pltpu.VMEM((4, 128), jnp.float32),
        pltpu.VMEM((TAIL * A_DIM,), jnp.float32),
        pltpu.VMEM((1024,), jnp.float32),
        pltpu.VMEM((L,), jnp.int32),
        pltpu.VMEM((L,), jnp.float32),
        pltpu.SemaphoreType.DMA,
    ],
    compiler_params=_params,
)
def _passA(ns_h, qtail_h, qT, qn_out,
           qn_sh, buf, nsv, own_ns, sidx, sval, tailv, zbuf, tidx, tval, sem):
    w = _wid()
    cid = lax.axis_index("c")
    sid = lax.axis_index("s")
    pltpu.sync_copy(ns_h, nsv)

    lo_b, hi_b = _region(w)
    lo_s, hi_s = lo_b * 128, hi_b * 128

    def build(b, cur):
        o = b * L
        ns = nsv[pl.ds(o, L)]
        mn = (ns >= lo_s) & (ns < hi_s)
        plsc.store_compressed(own_ns.at[pl.ds(cur, L)], o + _iota16(),
                              mask=mn)
        return cur + _scalar(plsc.all_reduce_population_count(mn))
    cnt_ns = lax.fori_loop(0, NBLK, build, 0)

    def z(b, _):
        zbuf[pl.ds(b * L, L)] = jnp.zeros((L,), jnp.float32)
        return 0
    lax.fori_loop(0, 1024 // L, z, 0)
    pltpu.sync_copy(zbuf, qn_sh.at[pl.ds(sid * 1024, 1024)])
    plsc.subcore_barrier()

    zero16 = jnp.zeros((L,), jnp.float32)
    zero16i = jnp.zeros((L,), jnp.int32)
    tn32 = ((cnt_ns + (32 * L - 1)) // (32 * L)) * 32

    def chunk(c, _):
        s0, can_lo, can_hi = _chunk_bounds(c, lo_b, hi_b, SA_BLK, TRIPS_A)
        pltpu.sync_copy(qT.at[pl.ds(0, A_DIM), pl.ds(s0, WA)], buf)

        def nblk_body(b, _):
            g = (b % 32) // 8
            r = (b % 8) * L
            ids = jnp.where((b * L + _iota16()) < cnt_ns,
                            own_ns[pl.ds(b * L, L)], 0)
            ns = plsc.load_gather(nsv, [ids])
            msk = ((b * L + _iota16()) < cnt_ns) \
                & (ns >= can_lo) & (ns < can_hi)

            def active(_):
                col = jnp.where(msk, ns - s0, 0)
                m = plsc.load_gather(buf, [zero16i, col])
                for a in range(1, A_DIM):
                    m = jnp.maximum(m, plsc.load_gather(
                        buf, [jnp.full((L,), a, jnp.int32), col]))
                sidx[g, pl.ds(r, L)] = jnp.where(msk, ids, 0)
                sval[g, pl.ds(r, L)] = jnp.where(msk, m, 0.0)
                return 0

            def idle(_):
                sidx[g, pl.ds(r, L)] = zero16i
                sval[g, pl.ds(r, L)] = zero16
                return 0

            lax.cond(jnp.any(msk), active, idle, 0)

            @pl.when(b % 32 == 31)
            def _():
                cps = [pltpu.async_copy(sval.at[j], qn_sh.at[sidx.at[j]],
                                        sem, add=True) for j in range(4)]
                for cp in cps:
                    cp.wait()
            return 0

        lax.fori_loop(0, tn32, nblk_body, 0)
        return 0

    lax.fori_loop(0, TRIPS_A, chunk, 0)

    @pl.when(w == 0)
    def _tail():
        pltpu.sync_copy(qtail_h, tailv)

        def blk(b, _):
            o = b * L
            ids = o + _iota16()
            ns = nsv[pl.ds(o, L)]
            mn = ns >= MAIN

            def do_ns(_):
                col = jnp.where(mn, (ns - MAIN) * A_DIM, 0)
                m = plsc.load_gather(tailv, [col])
                for a in range(1, A_DIM):
                    m = jnp.maximum(m, plsc.load_gather(tailv, [col + a]))
                tidx[pl.ds(0, L)] = jnp.where(mn, ids, 0)
                tval[pl.ds(0, L)] = jnp.where(mn, m, 0.0)
                pltpu.sync_copy(tval, qn_sh.at[tidx], add=True)
                return 0
            lax.cond(jnp.any(mn), do_ns, lambda _: 0, 0)
            return 0

        lax.fori_loop(0, NBLK, blk, 0)

    plsc.subcore_barrier()
    pltpu.sync_copy(qn_sh.at[pl.ds(sid * 1024, 1024)],
                    qn_out.at[pl.ds(cid * B + sid * 1024, 1024)])


PB = B // NW


@functools.partial(
    pl.kernel,
    out_type=(jax.ShapeDtypeStruct((B,), jnp.int32),
              jax.ShapeDtypeStruct((B,), jnp.float32)),
    mesh=_mesh,
    scratch_types=[
        pltpu.VMEM((PB,), jnp.int32),
        pltpu.VMEM((PB,), jnp.int32),
        pltpu.VMEM((PB,), jnp.float32),
        pltpu.VMEM((PB,), jnp.float32),
        pltpu.VMEM((PB,), jnp.float32),
        pltpu.VMEM((PB,), jnp.int32),
        pltpu.VMEM((PB,), jnp.float32),
    ],
    compiler_params=_params,
)
def _passB1(st_h, ac_h, rw_h, qn2_h, key_out, tdt_out,
            stv, acv, rwv, qna, qnb, kv, tv):
    w = _wid()
    base = w * PB
    pltpu.sync_copy(st_h.at[pl.ds(base, PB)], stv)
    pltpu.sync_copy(ac_h.at[pl.ds(base, PB)], acv)
    pltpu.sync_copy(rw_h.at[pl.ds(base, PB)], rwv)
    pltpu.sync_copy(qn2_h.at[pl.ds(base, PB)], qna)
    pltpu.sync_copy(qn2_h.at[pl.ds(B + base, PB)], qnb)

    def blk(b, _):
        o = b * L
        qn = qna[pl.ds(o, L)] + qnb[pl.ds(o, L)]
        tv[pl.ds(o, L)] = rwv[pl.ds(o, L)] + qn
        kv[pl.ds(o, L)] = stv[pl.ds(o, L)] * A_DIM + acv[pl.ds(o, L)]
        return 0
    lax.fori_loop(0, PB // L, blk, 0)

    pltpu.sync_copy(kv, key_out.at[pl.ds(base, PB)])
    pltpu.sync_copy(tv, tdt_out.at[pl.ds(base, PB)])


@functools.partial(
    pl.kernel,
    out_type=(jax.ShapeDtypeStruct((A_DIM, S_DIM), jnp.float32),
              jax.ShapeDtypeStruct((TAIL * A_DIM,), jnp.float32)),
    mesh=_mesh,
    scratch_types=[
        pltpu.VMEM((A_DIM, WB), jnp.float32),
        pltpu.VMEM((B,), jnp.int32),
        pltpu.VMEM((B,), jnp.float32),
        pltpu.VMEM((CAP,), jnp.int32),
        pltpu.VMEM((TAIL * A_DIM,), jnp.float32),
        pltpu.VMEM((L,), jnp.int32),
        pltpu.VMEM((L,), jnp.float32),
        pltpu.VMEM((L,), jnp.float32),
    ],
    compiler_params=_params,
)
def _passB2(qT, qtail_h, key_h, tdt_h, outT, tail_out, buf, keyv, tdtv,
            own_st, tailv, ksr, csr, vsr):
    w = _wid()
    pltpu.sync_copy(key_h, keyv)
    pltpu.sync_copy(tdt_h, tdtv)

    lo_b, hi_b = _region(w)
    klo0, khi0 = lo_b * 128 * A_DIM, hi_b * 128 * A_DIM

    def build(b, cur):
        o = b * L
        key = keyv[pl.ds(o, L)]
        msk = (key >= klo0) & (key < khi0)
        plsc.store_compressed(own_st.at[pl.ds(cur, L)], o + _iota16(),
                              mask=msk)
        return cur + _scalar(plsc.all_reduce_population_count(msk))
    cnt = lax.fori_loop(0, NBLK, build, 0)
    nblk = (cnt + L - 1) // L

    def chunk(c, _):
        s0, can_lo, can_hi = _chunk_bounds(c, lo_b, hi_b, SB_BLK, TRIPS_B)
        pltpu.sync_copy(qT.at[pl.ds(0, A_DIM), pl.ds(s0, WB)], buf)
        klo, khi = can_lo * A_DIM, can_hi * A_DIM

        def read_blk(b, _):
            pos = b * L + _iota16()
            live = pos < cnt
            ids = jnp.where(live, own_st[pl.ds(b * L, L)], 0)
            key = plsc.load_gather(keyv, [ids])
            msk = live & (key >= klo) & (key < khi)

            def active(_):
                st = lax.shift_right_logical(key, 5)
                ac = key & (A_DIM - 1)
                col = jnp.where(msk, st - s0, 0)
                qsa = plsc.load_gather(buf, [ac, col])
                tdt = plsc.load_gather(tdtv, [ids])
                plsc.store_scatter(tdtv, [ids], LR * (tdt - qsa), mask=msk)
                return 0
            lax.cond(jnp.any(msk), active, lambda _: 0, 0)
            return 0

        lax.fori_loop(0, nblk, read_blk, 0)

        def wr_blk(b, _):
            pos = b * L + _iota16()
            live = pos < cnt
            ids = jnp.where(live, own_st[pl.ds(b * L, L)], 0)
            key = plsc.load_gather(keyv, [ids])
            msk = live & (key >= klo) & (key < khi)

            def active(_):
                d = plsc.load_gather(tdtv, [ids])
                st = lax.shift_right_logical(key, 5)
                ac = key & (A_DIM - 1)
                fidx = ac * WB + (st - s0)
                ks, tot, ok = _dedup_tot(fidx, d, msk, ksr, csr, vsr)
                q, r = ks // WB, ks % WB
                plsc.addupdate_scatter(
                    buf, [jnp.where(ok, q, 0), jnp.where(ok, r, 0)],
                    tot, mask=ok)
                return 0
            lax.cond(jnp.any(msk), active, lambda _: 0, 0)
            return 0

        lax.fori_loop(0, nblk, wr_blk, 0)
        pltpu.sync_copy(buf, outT.at[pl.ds(0, A_DIM), pl.ds(s0, WB)])
        return 0

    lax.fori_loop(0, TRIPS_B, chunk, 0)

    @pl.when(w == 0)
    def _tail():
        pltpu.sync_copy(qtail_h, tailv)

        def read_blk(b, _):
            o = b * L
            key = keyv[pl.ds(o, L)]
            msk = key >= MAIN * A_DIM

            def active(_):
                idx = jnp.where(msk, key - MAIN * A_DIM, 0)
                qsa = plsc.load_gather(tailv, [idx])
                tdt = tdtv[pl.ds(o, L)]
                tdtv[pl.ds(o, L)] = jnp.where(msk, LR * (tdt - qsa), tdt)
                return 0
            lax.cond(jnp.any(msk), active, lambda _: 0, 0)
            return 0

        lax.fori_loop(0, NBLK, read_blk, 0)

        def wr_blk(b, _):
            key = keyv[pl.ds(b * L, L)]
            msk = key >= MAIN * A_DIM

            def active(_):
                d = tdtv[pl.ds(b * L, L)]
                fidx = key - MAIN * A_DIM
                ks, tot, ok = _dedup_tot(fidx, d, msk, ksr, csr, vsr)
                plsc.addupdate_scatter(tailv, [jnp.where(ok, ks, 0)], tot,
                                       mask=ok)
                return 0
            lax.cond(jnp.any(msk), active, lambda _: 0, 0)
            return 0

        lax.fori_loop(0, NBLK, wr_blk, 0)
        pltpu.sync_copy(tailv, tail_out)


def kernel(state, action, reward, next_state, instruction, q_table):
    del instruction
    st = state.astype(jnp.int32)
    ac = action.astype(jnp.int32)
    ns = next_state.astype(jnp.int32)
    qT = q_table.T
    qtail = q_table[MAIN:, :].reshape(-1)
    qn2 = _passA(ns, qtail, qT)
    keys, tdt = _passB1(st, ac, reward, qn2)
    outT, tail_out = _passB2(qT, qtail, keys, tdt)
    out = lax.dynamic_update_slice(outT.T, tail_out.reshape(TAIL, A_DIM),
                                   (MAIN, 0))
    return out
